# baseline (speedup 1.0000x reference)
import functools

import jax
import jax.numpy as jnp
from jax import lax
from jax.experimental import pallas as pl
from jax.experimental.pallas import tpu as pltpu
from jax.experimental.pallas import tpu_sc as plsc

NC, NS, L = 2, 16, 16
NW = NC * NS


def kernel(idx, wte, wpe):
    B, T = idx.shape
    V, D = wte.shape
    n_rows = B * T
    p_per_w = T // NW
    CHUNK = 32
    n_sub = p_per_w // CHUNK
    n_chunks = B * n_sub

    idx_flat = idx.reshape(n_rows).astype(jnp.int32)

    mesh = plsc.VectorSubcoreMesh(
        core_axis_name="c", subcore_axis_name="s",
        num_cores=NC, num_subcores=NS)

    @functools.partial(
        pl.kernel,
        out_type=jax.ShapeDtypeStruct((n_rows, D), jnp.float32),
        mesh=mesh,
        scratch_types=[
            pltpu.VMEM((n_chunks, CHUNK), jnp.int32),
            pltpu.VMEM((p_per_w, D), jnp.float32),
            pltpu.VMEM((2, CHUNK, D), jnp.float32),
            pltpu.SemaphoreType.DMA,
            pltpu.SemaphoreType.DMA,
            pltpu.SemaphoreType.DMA,
            pltpu.SemaphoreType.DMA,
        ],
    )
    def emb_kernel(idx_hbm, wte_hbm, wpe_hbm, out_hbm,
                   idx_v, wpe_v, rows_v, g0, g1, s0, s1):
        w = lax.axis_index("s") * NC + lax.axis_index("c")
        pos0 = w * p_per_w
        gsems = (g0, g1)
        ssems = (s0, s1)

        pltpu.sync_copy(wpe_hbm.at[pl.ds(pos0, p_per_w)], wpe_v)

        def chunk_base(c):
            b, sub = c // n_sub, c % n_sub
            return b * T + pos0 + sub * CHUNK

        for c in range(n_chunks):
            pltpu.sync_copy(idx_hbm.at[pl.ds(chunk_base(c), CHUNK)],
                            idx_v.at[c])

        def start_gather(c):
            s = c % 2
            return pltpu.async_copy(
                wte_hbm.at[idx_v.at[c]], rows_v.at[s], gsems[s])

        def start_store(c):
            s = c % 2
            return pltpu.async_copy(
                rows_v.at[s], out_hbm.at[pl.ds(chunk_base(c), CHUNK)],
                ssems[s])

        gathers = [None, None]
        stores = [None, None]
        gathers[0] = start_gather(0)
        for c in range(n_chunks):
            s = c % 2
            if c + 1 < n_chunks:
                s2 = (c + 1) % 2
                if stores[s2] is not None:
                    stores[s2].wait()
                gathers[s2] = start_gather(c + 1)
            gathers[s].wait()

            buf = rows_v.at[s]
            wpe_off = (c % n_sub) * CHUNK

            def add_body(r, _, buf=buf, wpe_off=wpe_off):
                for j in range(D // L):
                    sl = pl.ds(j * L, L)
                    buf[r, sl] = buf[r, sl] + wpe_v[wpe_off + r, sl]
                return _

            lax.fori_loop(0, CHUNK, add_body, None)
            stores[s] = start_store(c)

        stores[0].wait()
        stores[1].wait()

    out = emb_kernel(idx_flat, wte, wpe)
    return out.reshape(B, T, D)


# speedup vs baseline: 1.1552x; 1.1552x over previous
import functools

import jax
import jax.numpy as jnp
from jax import lax
from jax.experimental import pallas as pl
from jax.experimental.pallas import tpu as pltpu
from jax.experimental.pallas import tpu_sc as plsc

NC, NS, L = 2, 16, 16
NW = NC * NS
NBUF = 3


def kernel(idx, wte, wpe):
    B, T = idx.shape
    V, D = wte.shape
    p_per_w = T // NW
    CHUNK = 32
    n_sub = p_per_w // CHUNK
    n_chunks = B * n_sub

    idx = idx.astype(jnp.int32)

    mesh = plsc.VectorSubcoreMesh(
        core_axis_name="c", subcore_axis_name="s",
        num_cores=NC, num_subcores=NS)

    @functools.partial(
        pl.kernel,
        out_type=jax.ShapeDtypeStruct((B, T, D), jnp.float32),
        mesh=mesh,
        scratch_types=[
            pltpu.VMEM((n_chunks, CHUNK), jnp.int32),
            pltpu.VMEM((p_per_w, D), jnp.float32),
            pltpu.VMEM((NBUF, CHUNK, D), jnp.float32),
        ] + [pltpu.SemaphoreType.DMA] * (2 * NBUF),
    )
    def emb_kernel(idx_hbm, wte_hbm, wpe_hbm, out_hbm,
                   idx_v, wpe_v, rows_v, *sems):
        w = lax.axis_index("s") * NC + lax.axis_index("c")
        pos0 = w * p_per_w
        gsems = sems[:NBUF]
        ssems = sems[NBUF:]

        pltpu.sync_copy(wpe_hbm.at[pl.ds(pos0, p_per_w)], wpe_v)

        def chunk_batch_off(c):
            return c // n_sub, pos0 + (c % n_sub) * CHUNK

        for c in range(n_chunks):
            b, off = chunk_batch_off(c)
            pltpu.sync_copy(idx_hbm.at[b, pl.ds(off, CHUNK)], idx_v.at[c])

        def start_gather(c):
            s = c % NBUF
            return pltpu.async_copy(
                wte_hbm.at[idx_v.at[c]], rows_v.at[s], gsems[s])

        def start_store(c):
            s = c % NBUF
            b, off = chunk_batch_off(c)
            return pltpu.async_copy(
                rows_v.at[s], out_hbm.at[b, pl.ds(off, CHUNK)], ssems[s])

        gathers = [None] * NBUF
        stores = [None] * NBUF
        for c in range(NBUF - 1):
            gathers[c % NBUF] = start_gather(c)
        for c in range(n_chunks):
            s = c % NBUF
            cn = c + NBUF - 1
            if cn < n_chunks:
                s2 = cn % NBUF
                if stores[s2] is not None:
                    stores[s2].wait()
                gathers[s2] = start_gather(cn)
            gathers[s].wait()

            buf = rows_v.at[s]
            wpe_off = (c % n_sub) * CHUNK

            def add_body(r, _, buf=buf, wpe_off=wpe_off):
                for j in range(D // L):
                    sl = pl.ds(j * L, L)
                    plsc.addupdate(buf.at[r, sl], wpe_v[wpe_off + r, sl])
                return _

            lax.fori_loop(0, CHUNK, add_body, None)
            stores[s] = start_store(c)

        for st in stores:
            if st is not None:
                st.wait()

    return emb_kernel(idx, wte, wpe)


# speedup vs baseline: 1.2451x; 1.0778x over previous
import functools

import jax
import jax.numpy as jnp
from jax import lax
from jax.experimental import pallas as pl
from jax.experimental.pallas import tpu as pltpu
from jax.experimental.pallas import tpu_sc as plsc

NC, NS, L = 2, 16, 16
NW = NC * NS
NBUF = 3


def kernel(idx, wte, wpe):
    B, T = idx.shape
    V, D = wte.shape
    p_per_w = T // NW
    CHUNK = 32
    n_sub = p_per_w // CHUNK
    n_chunks = B * n_sub

    idx = idx.astype(jnp.int32)

    mesh = plsc.VectorSubcoreMesh(
        core_axis_name="c", subcore_axis_name="s",
        num_cores=NC, num_subcores=NS)

    @functools.partial(
        pl.kernel,
        out_type=jax.ShapeDtypeStruct((B, T, D), jnp.float32),
        mesh=mesh,
        scratch_types=[
            pltpu.VMEM((n_chunks, CHUNK), jnp.int32),
            pltpu.VMEM((p_per_w, D), jnp.float32),
            pltpu.VMEM((NBUF, CHUNK, D), jnp.float32),
        ] + [pltpu.SemaphoreType.DMA] * (2 * NBUF + 2),
    )
    def emb_kernel(idx_hbm, wte_hbm, wpe_hbm, out_hbm,
                   idx_v, wpe_v, rows_v, *sems):
        w = lax.axis_index("s") * NC + lax.axis_index("c")
        pos0 = w * p_per_w
        gsems = sems[:NBUF]
        ssems = sems[NBUF:2 * NBUF]
        wsem, isem = sems[2 * NBUF], sems[2 * NBUF + 1]

        def chunk_batch_off(c):
            return c // n_sub, pos0 + (c % n_sub) * CHUNK

        wpe_cp = pltpu.async_copy(wpe_hbm.at[pl.ds(pos0, p_per_w)],
                                  wpe_v, wsem)
        idx_cps = []
        for c in range(n_chunks):
            b, off = chunk_batch_off(c)
            idx_cps.append(pltpu.async_copy(
                idx_hbm.at[b, pl.ds(off, CHUNK)], idx_v.at[c], isem))
        for cp in idx_cps:
            cp.wait()

        def start_gather(c):
            s = c % NBUF
            return pltpu.async_copy(
                wte_hbm.at[idx_v.at[c]], rows_v.at[s], gsems[s])

        def start_store(c):
            s = c % NBUF
            b, off = chunk_batch_off(c)
            return pltpu.async_copy(
                rows_v.at[s], out_hbm.at[b, pl.ds(off, CHUNK)], ssems[s])

        gathers = [None] * NBUF
        stores = [None] * NBUF
        for c in range(NBUF - 1):
            gathers[c % NBUF] = start_gather(c)
        wpe_cp.wait()
        for c in range(n_chunks):
            s = c % NBUF
            cn = c + NBUF - 1
            if cn < n_chunks:
                s2 = cn % NBUF
                if stores[s2] is not None:
                    stores[s2].wait()
                gathers[s2] = start_gather(cn)
            gathers[s].wait()

            buf = rows_v.at[s]
            wpe_off = (c % n_sub) * CHUNK

            def add_body(r, _, buf=buf, wpe_off=wpe_off):
                for j in range(D // L):
                    sl = pl.ds(j * L, L)
                    plsc.addupdate(buf.at[r, sl], wpe_v[wpe_off + r, sl])
                return _

            lax.fori_loop(0, CHUNK, add_body, None)
            stores[s] = start_store(c)

        for st in stores:
            if st is not None:
                st.wait()

    return emb_kernel(idx, wte, wpe)


# speedup vs baseline: 1.5126x; 1.2148x over previous
import functools

import jax
import jax.numpy as jnp
from jax import lax
from jax.experimental import pallas as pl
from jax.experimental.pallas import tpu as pltpu
from jax.experimental.pallas import tpu_sc as plsc

NC, NS, L = 2, 16, 16
NW = NC * NS
NBUF = 3
P = 8


def kernel(idx, wte, wpe):
    B, T = idx.shape
    V, D = wte.shape
    p_per_w = T // NW
    n_chunks = p_per_w // P

    idx = idx.astype(jnp.int32)

    mesh = plsc.VectorSubcoreMesh(
        core_axis_name="c", subcore_axis_name="s",
        num_cores=NC, num_subcores=NS)

    @functools.partial(
        pl.kernel,
        out_type=jax.ShapeDtypeStruct((B, T, D), jnp.float32),
        mesh=mesh,
        scratch_types=[
            pltpu.VMEM((B, p_per_w), jnp.int32),
            pltpu.VMEM((p_per_w, D), jnp.float32),
            pltpu.VMEM((NBUF, B, P, D), jnp.float32),
        ] + [pltpu.SemaphoreType.DMA] * (2 * NBUF + 2),
    )
    def emb_kernel(idx_hbm, wte_hbm, wpe_hbm, out_hbm,
                   idx_v, wpe_v, rows_v, *sems):
        w = lax.axis_index("s") * NC + lax.axis_index("c")
        pos0 = w * p_per_w
        gsems = sems[:NBUF]
        ssems = sems[NBUF:2 * NBUF]
        wsem, isem = sems[2 * NBUF], sems[2 * NBUF + 1]

        wpe_cp = pltpu.async_copy(wpe_hbm.at[pl.ds(pos0, p_per_w)],
                                  wpe_v, wsem)
        idx_cps = [
            pltpu.async_copy(idx_hbm.at[b, pl.ds(pos0, p_per_w)],
                             idx_v.at[b], isem)
            for b in range(B)
        ]
        for cp in idx_cps:
            cp.wait()

        def start_gathers(c):
            s = c % NBUF
            return [
                pltpu.async_copy(
                    wte_hbm.at[idx_v.at[b, pl.ds(c * P, P)]],
                    rows_v.at[s, b], gsems[s])
                for b in range(B)
            ]

        def start_stores(c):
            s = c % NBUF
            return [
                pltpu.async_copy(
                    rows_v.at[s, b],
                    out_hbm.at[b, pl.ds(pos0 + c * P, P)], ssems[s])
                for b in range(B)
            ]

        gathers = [None] * NBUF
        stores = [None] * NBUF
        for c in range(NBUF - 1):
            gathers[c % NBUF] = start_gathers(c)
        wpe_cp.wait()
        for c in range(n_chunks):
            s = c % NBUF
            cn = c + NBUF - 1
            if cn < n_chunks:
                s2 = cn % NBUF
                if stores[s2] is not None:
                    for cp in stores[s2]:
                        cp.wait()
                gathers[s2] = start_gathers(cn)
            for cp in gathers[s]:
                cp.wait()

            buf = rows_v.at[s]

            def add_body(p, _, buf=buf, c=c):
                wrow = [wpe_v[c * P + p, pl.ds(j * L, L)]
                        for j in range(D // L)]
                for b in range(B):
                    for j in range(D // L):
                        plsc.addupdate(buf.at[b, p, pl.ds(j * L, L)],
                                       wrow[j])
                return _

            lax.fori_loop(0, P, add_body, None)
            stores[s] = start_stores(c)

        for st in stores:
            if st is not None:
                for cp in st:
                    cp.wait()

    return emb_kernel(idx, wte, wpe)
